# Initial kernel scaffold; baseline (speedup 1.0000x reference)
import jax
import jax.numpy as jnp
from jax.experimental import pallas as pl


def kernel(x, edge_index, batch, Wp, bp, g0, b0, Wl, bl, Wr, br, att, bgat, W1, b1, g1, bb1, lng, lnb, Wres, bres, Wg1, bg1, Wg2, bg2, Wh1, bh1, gh, bh, Wh2, bh2):
    raise NotImplementedError("write your pallas kernel here")


# speedup vs baseline: 14.6245x; 14.6245x over previous
import functools

import jax
import jax.numpy as jnp
from jax import lax
from jax.experimental import pallas as pl
from jax.experimental.pallas import tpu as pltpu
from jax.experimental.pallas import tpu_sc as plsc

N = 10000
E = 320000
G = 64
D = 128
H = 4
DH = 32
O = 8

NC = 2
NS = 16
NW = NC * NS
EPW = E // NW
KA = 80
NCHA = EPW // KA
KB = 40
NCHB = EPW // KB
KC = 80
NCHC = EPW // KC
NR = 10240
STRIPE = NR // NS

_f32 = jnp.float32
_i32 = jnp.int32
_mesh = plsc.VectorSubcoreMesh(core_axis_name="c", subcore_axis_name="s")
_sc_params = pltpu.CompilerParams(needs_layout_passes=False)


def _mmT(a, w):
    return lax.dot_general(a, w, (((1,), (1,)), ((), ())),
                           preferred_element_type=jnp.float32,
                           precision=lax.Precision.HIGHEST)


def _tc_proj(x, Wp, bp, g0, b0, Wl, bl, Wr, br):
    BLK = 2000

    def body(x_r, wp_r, bp_r, g0_r, b0_r, wl_r, bl_r, wr_r, br_r,
             h0_r, xl_r, xr_r):
        h = _mmT(x_r[...], wp_r[...]) + bp_r[...]
        h = jnp.maximum(h, 0.0)
        h = h * g0_r[...] + b0_r[...]
        h0_r[...] = h
        xl_r[...] = _mmT(h, wl_r[...]) + bl_r[...]
        xr_r[...] = _mmT(h, wr_r[...]) + br_r[...]

    row_spec = pl.BlockSpec((BLK, D), lambda i: (i, 0))
    mat_spec = pl.BlockSpec((D, D), lambda i: (0, 0))
    vec_spec = pl.BlockSpec((D,), lambda i: (0,))
    return pl.pallas_call(
        body,
        grid=(N // BLK,),
        in_specs=[row_spec, mat_spec, vec_spec, vec_spec, vec_spec,
                  mat_spec, vec_spec, mat_spec, vec_spec],
        out_specs=[row_spec, row_spec, row_spec],
        out_shape=[jax.ShapeDtypeStruct((N, D), _f32)] * 3,
    )(x, Wp, bp, g0, b0, Wl, bl, Wr, br)


def _sc_scores(xl, xr, src, dst, att_flat):
    @functools.partial(
        pl.kernel,
        mesh=_mesh,
        compiler_params=_sc_params,
        out_type=[
            jax.ShapeDtypeStruct((E * H,), _f32),
            jax.ShapeDtypeStruct((NW, H * N), _f32),
        ],
        scratch_types=[
            pltpu.VMEM((KA, D), _f32),
            pltpu.VMEM((KA, D), _f32),
            pltpu.VMEM((KA,), _i32),
            pltpu.VMEM((KA,), _i32),
            pltpu.VMEM((D,), _f32),
            pltpu.VMEM((EPW * H,), _f32),
            pltpu.VMEM((H * N,), _f32),
            pltpu.SemaphoreType.DMA,
            pltpu.SemaphoreType.DMA,
        ],
    )
    def k(xl_hbm, xr_hbm, src_hbm, dst_hbm, att_hbm, ex_hbm, den_hbm,
          xl_rows, xr_rows, src_v, dst_v, att_v, ex_v, den_local,
          sem1, sem2):
        c = lax.axis_index("c")
        s = lax.axis_index("s")
        wid = c * NS + s
        wbase = wid * EPW

        def zero_body(i, _):
            den_local[pl.ds(i * 16, 16)] = jnp.zeros((16,), _f32)
            return 0
        lax.fori_loop(0, (H * N) // 16, zero_body, 0)

        pltpu.sync_copy(att_hbm, att_v)
        ii = lax.iota(_i32, 16)

        def chunk(ci, _):
            base = wbase + ci * KA
            pltpu.sync_copy(src_hbm.at[pl.ds(base, KA)], src_v)
            pltpu.sync_copy(dst_hbm.at[pl.ds(base, KA)], dst_v)
            cp1 = pltpu.async_copy(xl_hbm.at[src_v], xl_rows, sem1)
            cp2 = pltpu.async_copy(xr_hbm.at[dst_v], xr_rows, sem2)
            cp1.wait()
            cp2.wait()

            def grp(g, _):
                rowid = g * 16 + ii
                dst16 = dst_v[pl.ds(g * 16, 16)]
                for h in range(H):
                    acc = jnp.zeros((16,), _f32)
                    for dd in range(DH):
                        d = h * DH + dd
                        col = jnp.full((16,), d, _i32)
                        a = (plsc.load_gather(xl_rows, [rowid, col])
                             + plsc.load_gather(xr_rows, [rowid, col]))
                        a = jnp.maximum(a, 0.2 * a)
                        acc = acc + a * plsc.load_gather(att_v, [col])
                    ex16 = jnp.exp(acc)
                    plsc.store_scatter(
                        ex_v, [(ci * KA + rowid) * H + h], ex16)
                    plsc.addupdate_scatter(den_local, [dst16 + h * N], ex16)
                return 0
            lax.fori_loop(0, KA // 16, grp, 0)
            return 0
        lax.fori_loop(0, NCHA, chunk, 0)

        pltpu.sync_copy(ex_v, ex_hbm.at[pl.ds(wbase * H, EPW * H)])
        pltpu.sync_copy(den_local, den_hbm.at[wid])

    return k(xl, xr, src, dst, att_flat)


def _tc_rden(den_partials):
    def body(d_r, out_r):
        out_r[...] = 1.0 / (jnp.sum(d_r[...], axis=0) + 1e-16)

    return pl.pallas_call(
        body,
        out_shape=jax.ShapeDtypeStruct((H * N,), _f32),
    )(den_partials)


def _sc_gat_agg(xl, src, dst, ex, rden):
    @functools.partial(
        pl.kernel,
        mesh=_mesh,
        compiler_params=_sc_params,
        out_type=jax.ShapeDtypeStruct((NC, NR, D), _f32),
        scratch_types=[
            pltpu.VMEM((KB, D), _f32),
            pltpu.VMEM((KB,), _i32),
            pltpu.VMEM((KB,), _i32),
            pltpu.VMEM((KB * H,), _f32),
            pltpu.VMEM((KB * H,), _f32),
            pltpu.VMEM((H * N,), _f32),
            pltpu.VMEM_SHARED((NR, D), _f32),
            pltpu.SemaphoreType.DMA,
        ],
    )
    def k(xl_hbm, src_hbm, dst_hbm, ex_hbm, rden_hbm, out_hbm,
          rows, src_v, dst_v, ex_c, alpha_c, rden_v, acc_sh, sem1):
        c = lax.axis_index("c")
        s = lax.axis_index("s")
        wid = c * NS + s
        wbase = wid * EPW

        pltpu.sync_copy(rden_hbm, rden_v)

        def zero_body(i, _):
            for j in range(8):
                rows[i, pl.ds(j * 16, 16)] = jnp.zeros((16,), _f32)
            return 0
        lax.fori_loop(0, KB, zero_body, 0)
        stripe = s * STRIPE
        for r in range(STRIPE // KB):
            pltpu.sync_copy(rows, acc_sh.at[pl.ds(stripe + r * KB, KB)])
        plsc.subcore_barrier()

        ii = lax.iota(_i32, 16)
        erep = ii // H
        hrep = ii - erep * H

        def chunk(ci, _):
            base = wbase + ci * KB
            pltpu.sync_copy(src_hbm.at[pl.ds(base, KB)], src_v)
            pltpu.sync_copy(dst_hbm.at[pl.ds(base, KB)], dst_v)
            cp = pltpu.async_copy(xl_hbm.at[src_v], rows, sem1)
            pltpu.sync_copy(ex_hbm.at[pl.ds(base * H, KB * H)], ex_c)
            cp.wait()

            def grp(g, _):
                dst4 = plsc.load_gather(dst_v, [g * 4 + erep])
                rv = plsc.load_gather(rden_v, [dst4 + hrep * N])
                alpha_c[pl.ds(g * 16, 16)] = ex_c[pl.ds(g * 16, 16)] * rv
                return 0
            lax.fori_loop(0, (KB * H) // 16, grp, 0)

            def edge(e, _):
                for h in range(H):
                    av = plsc.load_gather(
                        alpha_c, [jnp.full((16,), e * H + h, _i32)])
                    j0 = h * 32
                    rows[e, pl.ds(j0, 16)] = rows[e, pl.ds(j0, 16)] * av
                    rows[e, pl.ds(j0 + 16, 16)] = (
                        rows[e, pl.ds(j0 + 16, 16)] * av)
                return 0
            lax.fori_loop(0, KB, edge, 0)

            pltpu.sync_copy(rows, acc_sh.at[dst_v], add=True)
            return 0
        lax.fori_loop(0, NCHB, chunk, 0)

        plsc.subcore_barrier()
        pltpu.sync_copy(acc_sh.at[pl.ds(stripe, STRIPE)],
                        out_hbm.at[c, pl.ds(stripe, STRIPE)])

    return k(xl, src, dst, ex, rden)


def _tc_combine(parts, bias):
    def body(p_r, b_r, out_r):
        out_r[...] = p_r[0, :N, :] + p_r[1, :N, :] + b_r[...]

    return pl.pallas_call(
        body,
        out_shape=jax.ShapeDtypeStruct((N, D), _f32),
    )(parts, bias)


def _sc_gin_agg(h, src, dst):
    @functools.partial(
        pl.kernel,
        mesh=_mesh,
        compiler_params=_sc_params,
        out_type=jax.ShapeDtypeStruct((NC, NR, D), _f32),
        scratch_types=[
            pltpu.VMEM((KC, D), _f32),
            pltpu.VMEM((KC,), _i32),
            pltpu.VMEM((KC,), _i32),
            pltpu.VMEM_SHARED((NR, D), _f32),
            pltpu.SemaphoreType.DMA,
        ],
    )
    def k(h_hbm, src_hbm, dst_hbm, out_hbm, rows, src_v, dst_v, acc_sh, sem1):
        c = lax.axis_index("c")
        s = lax.axis_index("s")
        wid = c * NS + s
        wbase = wid * EPW

        def zero_body(i, _):
            for j in range(8):
                rows[i, pl.ds(j * 16, 16)] = jnp.zeros((16,), _f32)
            return 0
        lax.fori_loop(0, KC, zero_body, 0)
        stripe = s * STRIPE
        for r in range(STRIPE // KC):
            pltpu.sync_copy(rows, acc_sh.at[pl.ds(stripe + r * KC, KC)])
        plsc.subcore_barrier()

        def chunk(ci, _):
            base = wbase + ci * KC
            pltpu.sync_copy(src_hbm.at[pl.ds(base, KC)], src_v)
            pltpu.sync_copy(dst_hbm.at[pl.ds(base, KC)], dst_v)
            pltpu.async_copy(h_hbm.at[src_v], rows, sem1).wait()
            pltpu.sync_copy(rows, acc_sh.at[dst_v], add=True)
            return 0
        lax.fori_loop(0, NCHC, chunk, 0)

        plsc.subcore_barrier()
        pltpu.sync_copy(acc_sh.at[pl.ds(stripe, STRIPE)],
                        out_hbm.at[c, pl.ds(stripe, STRIPE)])

    return k(h, src, dst)


def _tc_tail(h_gat, gin_parts, x_res, batch2d, W1, b1, g1, bb1, lng, lnb,
             Wres, bres, Wg1, bg1, Wg2, bg2, Wh1, bh1, gh, bh, Wh2, bh2):
    def body(hg_r, gp_r, xres_r, batch_r, w1_r, b1_r, g1_r, bb1_r, lng_r,
             lnb_r, wres_r, bres_r, wg1_r, bg1_r, wg2_r, bg2_r, wh1_r, bh1_r,
             gh_r, bh_r, wh2_r, bh2_r, out_r):
        h = hg_r[...] + gp_r[0, :N, :] + gp_r[1, :N, :]
        h = _mmT(h, w1_r[...]) + b1_r[...]
        h = jnp.maximum(h, 0.0)
        h = h * g1_r[...] + bb1_r[...]
        h = h - jnp.mean(h)
        hc = h - jnp.mean(h)
        std = jnp.sqrt(jnp.mean(hc * hc))
        h = h / (std + 1e-5)
        h = h * lng_r[...] + lnb_r[...]
        h = h + _mmT(xres_r[...], wres_r[...]) + bres_r[...]
        h = jnp.maximum(h, 0.2 * h)
        t = jnp.tanh(_mmT(h, wg1_r[...]) + bg1_r[...])
        gate = jnp.sum(t * wg2_r[...], axis=1, keepdims=True) + bg2_r[0]
        ge = jnp.exp(gate)
        onehot = (batch_r[...]
                  == lax.broadcasted_iota(_i32, (G, 1), 0)).astype(_f32)
        gden = lax.dot_general(onehot, ge, (((1,), (0,)), ((), ())),
                               preferred_element_type=_f32,
                               precision=lax.Precision.HIGHEST)
        u = lax.dot_general(onehot, ge * h, (((1,), (0,)), ((), ())),
                            preferred_element_type=_f32,
                            precision=lax.Precision.HIGHEST)
        emb = u / (gden + 1e-16)
        outs = []
        for o in range(O):
            z = _mmT(emb, wh1_r[o]) + bh1_r[o]
            z = z * jax.nn.sigmoid(z)
            z = z * gh_r[o] + bh_r[o]
            outs.append(jnp.sum(z * wh2_r[o], axis=1, keepdims=True)
                        + bh2_r[o, 0])
        out_r[...] = jnp.concatenate(outs, axis=1)

    return pl.pallas_call(
        body,
        out_shape=jax.ShapeDtypeStruct((G, O), _f32),
    )(h_gat, gin_parts, x_res, batch2d, W1, b1, g1, bb1, lng, lnb,
      Wres, bres, Wg1, bg1, Wg2, bg2, Wh1, bh1, gh, bh, Wh2, bh2)


def kernel(x, edge_index, batch, Wp, bp, g0, b0, Wl, bl, Wr, br, att, bgat,
           W1, b1, g1, bb1, lng, lnb, Wres, bres, Wg1, bg1, Wg2, bg2,
           Wh1, bh1, gh, bh, Wh2, bh2):
    src = edge_index[0]
    dst = edge_index[1]
    h0, xl, xr = _tc_proj(x, Wp, bp, g0, b0, Wl, bl, Wr, br)
    ex, den_parts = _sc_scores(xl, xr, src, dst, att.reshape(H * DH))
    rden = _tc_rden(den_parts)
    gat_parts = _sc_gat_agg(xl, src, dst, ex, rden)
    h_gat = _tc_combine(gat_parts, bgat)
    gin_parts = _sc_gin_agg(h_gat, src, dst)
    out = _tc_tail(h_gat, gin_parts, h0, batch.reshape(1, N), W1, b1, g1,
                   bb1, lng, lnb, Wres, bres, Wg1, bg1, Wg2, bg2,
                   Wh1, bh1, gh, bh, Wh2, bh2)
    return out.reshape(G, O, 1)


# speedup vs baseline: 25.6393x; 1.7532x over previous
import functools

import jax
import jax.numpy as jnp
from jax import lax
from jax.experimental import pallas as pl
from jax.experimental.pallas import tpu as pltpu
from jax.experimental.pallas import tpu_sc as plsc

N = 10000
E = 320000
G = 64
D = 128
H = 4
DH = 32
O = 8

NC = 2
NS = 16
NW = NC * NS
EPW = E // NW
KA = 80
NCHA = EPW // KA
KB = 40
NCHB = EPW // KB
KC = 80
NCHC = EPW // KC
NR = 10240
STRIPE = NR // NS

_f32 = jnp.float32
_i32 = jnp.int32
_mesh = plsc.VectorSubcoreMesh(core_axis_name="c", subcore_axis_name="s")
_sc_params = pltpu.CompilerParams(needs_layout_passes=False)


def _mmT(a, w):
    return lax.dot_general(a, w, (((1,), (1,)), ((), ())),
                           preferred_element_type=jnp.float32,
                           precision=lax.Precision.HIGHEST)


def _tc_proj(x, Wp, bp, g0, b0, Wl, bl, Wr, br):
    BLK = 2000

    def body(x_r, wp_r, bp_r, g0_r, b0_r, wl_r, bl_r, wr_r, br_r,
             h0_r, xl_r, xr_r):
        h = _mmT(x_r[...], wp_r[...]) + bp_r[...]
        h = jnp.maximum(h, 0.0)
        h = h * g0_r[...] + b0_r[...]
        h0_r[...] = h
        xl_r[...] = _mmT(h, wl_r[...]) + bl_r[...]
        xr_r[...] = _mmT(h, wr_r[...]) + br_r[...]

    row_spec = pl.BlockSpec((BLK, D), lambda i: (i, 0))
    mat_spec = pl.BlockSpec((D, D), lambda i: (0, 0))
    vec_spec = pl.BlockSpec((D,), lambda i: (0,))
    return pl.pallas_call(
        body,
        grid=(N // BLK,),
        in_specs=[row_spec, mat_spec, vec_spec, vec_spec, vec_spec,
                  mat_spec, vec_spec, mat_spec, vec_spec],
        out_specs=[row_spec, row_spec, row_spec],
        out_shape=[jax.ShapeDtypeStruct((N, D), _f32)] * 3,
    )(x, Wp, bp, g0, b0, Wl, bl, Wr, br)


def _sc_scores(xl, xr, src, dst, att_flat):
    @functools.partial(
        pl.kernel,
        mesh=_mesh,
        compiler_params=_sc_params,
        out_type=[
            jax.ShapeDtypeStruct((E * H,), _f32),
            jax.ShapeDtypeStruct((NW, H * N), _f32),
        ],
        scratch_types=[
            pltpu.VMEM((KA, D), _f32),
            pltpu.VMEM((KA, D), _f32),
            pltpu.VMEM((KA,), _i32),
            pltpu.VMEM((KA,), _i32),
            pltpu.VMEM((D,), _f32),
            pltpu.VMEM((EPW * H,), _f32),
            pltpu.VMEM((H * N,), _f32),
            pltpu.SemaphoreType.DMA,
            pltpu.SemaphoreType.DMA,
        ],
    )
    def k(xl_hbm, xr_hbm, src_hbm, dst_hbm, att_hbm, ex_hbm, den_hbm,
          xl_rows, xr_rows, src_v, dst_v, att_v, ex_v, den_local,
          sem1, sem2):
        c = lax.axis_index("c")
        s = lax.axis_index("s")
        wid = c * NS + s
        wbase = wid * EPW

        def zero_body(i, _):
            den_local[pl.ds(i * 16, 16)] = jnp.zeros((16,), _f32)
            return 0
        lax.fori_loop(0, (H * N) // 16, zero_body, 0)

        pltpu.sync_copy(att_hbm, att_v)
        ii = lax.iota(_i32, 16)
        p8, p4, p2, p1 = ii ^ 8, ii ^ 4, ii ^ 2, ii ^ 1
        attv = [att_v[pl.ds(j * 16, 16)] for j in range(8)]
        lmask = [ii == t for t in range(16)]
        zero16 = jnp.zeros((16,), _f32)

        def chunk(ci, _):
            base = wbase + ci * KA
            pltpu.sync_copy(src_hbm.at[pl.ds(base, KA)], src_v)
            pltpu.sync_copy(dst_hbm.at[pl.ds(base, KA)], dst_v)
            cp1 = pltpu.async_copy(xl_hbm.at[src_v], xl_rows, sem1)
            cp2 = pltpu.async_copy(xr_hbm.at[dst_v], xr_rows, sem2)
            cp1.wait()
            cp2.wait()

            def grp(g, _):
                dst16 = dst_v[pl.ds(g * 16, 16)]
                exs = [zero16] * H
                for t in range(16):
                    e = g * 16 + t
                    for h in range(H):
                        j0 = h * 32
                        a = (xl_rows[e, pl.ds(j0, 16)]
                             + xr_rows[e, pl.ds(j0, 16)])
                        b = (xl_rows[e, pl.ds(j0 + 16, 16)]
                             + xr_rows[e, pl.ds(j0 + 16, 16)])
                        a = jnp.maximum(a, 0.2 * a) * attv[2 * h]
                        b = jnp.maximum(b, 0.2 * b) * attv[2 * h + 1]
                        v = a + b
                        v = v + v[p8]
                        v = v + v[p4]
                        v = v + v[p2]
                        v = v + v[p1]
                        exs[h] = jnp.where(lmask[t], v, exs[h])
                for h in range(H):
                    ex16 = jnp.exp(exs[h])
                    plsc.store_scatter(
                        ex_v, [(ci * KA + g * 16 + ii) * H + h], ex16)
                    plsc.addupdate_scatter(den_local, [dst16 + h * N], ex16)
                return 0
            lax.fori_loop(0, KA // 16, grp, 0)
            return 0
        lax.fori_loop(0, NCHA, chunk, 0)

        pltpu.sync_copy(ex_v, ex_hbm.at[pl.ds(wbase * H, EPW * H)])
        pltpu.sync_copy(den_local, den_hbm.at[wid])

    return k(xl, xr, src, dst, att_flat)


def _tc_rden(den_partials):
    def body(d_r, out_r):
        out_r[...] = 1.0 / (jnp.sum(d_r[...], axis=0) + 1e-16)

    return pl.pallas_call(
        body,
        out_shape=jax.ShapeDtypeStruct((H * N,), _f32),
    )(den_partials)


def _sc_gat_agg(xl, src, dst, ex, rden):
    @functools.partial(
        pl.kernel,
        mesh=_mesh,
        compiler_params=_sc_params,
        out_type=jax.ShapeDtypeStruct((NC, NR, D), _f32),
        scratch_types=[
            pltpu.VMEM((KB, D), _f32),
            pltpu.VMEM((KB,), _i32),
            pltpu.VMEM((KB,), _i32),
            pltpu.VMEM((KB * H,), _f32),
            pltpu.VMEM((KB * H,), _f32),
            pltpu.VMEM((H * N,), _f32),
            pltpu.VMEM_SHARED((NR, D), _f32),
            pltpu.SemaphoreType.DMA,
        ],
    )
    def k(xl_hbm, src_hbm, dst_hbm, ex_hbm, rden_hbm, out_hbm,
          rows, src_v, dst_v, ex_c, alpha_c, rden_v, acc_sh, sem1):
        c = lax.axis_index("c")
        s = lax.axis_index("s")
        wid = c * NS + s
        wbase = wid * EPW

        pltpu.sync_copy(rden_hbm, rden_v)

        def zero_body(i, _):
            for j in range(8):
                rows[i, pl.ds(j * 16, 16)] = jnp.zeros((16,), _f32)
            return 0
        lax.fori_loop(0, KB, zero_body, 0)
        stripe = s * STRIPE
        for r in range(STRIPE // KB):
            pltpu.sync_copy(rows, acc_sh.at[pl.ds(stripe + r * KB, KB)])
        plsc.subcore_barrier()

        ii = lax.iota(_i32, 16)
        erep = ii // H
        hrep = ii - erep * H

        def chunk(ci, _):
            base = wbase + ci * KB
            pltpu.sync_copy(src_hbm.at[pl.ds(base, KB)], src_v)
            pltpu.sync_copy(dst_hbm.at[pl.ds(base, KB)], dst_v)
            cp = pltpu.async_copy(xl_hbm.at[src_v], rows, sem1)
            pltpu.sync_copy(ex_hbm.at[pl.ds(base * H, KB * H)], ex_c)
            cp.wait()

            def grp(g, _):
                dst4 = plsc.load_gather(dst_v, [g * 4 + erep])
                rv = plsc.load_gather(rden_v, [dst4 + hrep * N])
                alpha_c[pl.ds(g * 16, 16)] = ex_c[pl.ds(g * 16, 16)] * rv
                return 0
            lax.fori_loop(0, (KB * H) // 16, grp, 0)

            def edge(e, _):
                for h in range(H):
                    av = plsc.load_gather(
                        alpha_c, [jnp.full((16,), e * H + h, _i32)])
                    j0 = h * 32
                    rows[e, pl.ds(j0, 16)] = rows[e, pl.ds(j0, 16)] * av
                    rows[e, pl.ds(j0 + 16, 16)] = (
                        rows[e, pl.ds(j0 + 16, 16)] * av)
                return 0
            lax.fori_loop(0, KB, edge, 0)

            pltpu.sync_copy(rows, acc_sh.at[dst_v], add=True)
            return 0
        lax.fori_loop(0, NCHB, chunk, 0)

        plsc.subcore_barrier()
        pltpu.sync_copy(acc_sh.at[pl.ds(stripe, STRIPE)],
                        out_hbm.at[c, pl.ds(stripe, STRIPE)])

    return k(xl, src, dst, ex, rden)


def _tc_combine(parts, bias):
    def body(p_r, b_r, out_r):
        out_r[...] = p_r[0, :N, :] + p_r[1, :N, :] + b_r[...]

    return pl.pallas_call(
        body,
        out_shape=jax.ShapeDtypeStruct((N, D), _f32),
    )(parts, bias)


def _sc_gin_agg(h, src, dst):
    @functools.partial(
        pl.kernel,
        mesh=_mesh,
        compiler_params=_sc_params,
        out_type=jax.ShapeDtypeStruct((NC, NR, D), _f32),
        scratch_types=[
            pltpu.VMEM((KC, D), _f32),
            pltpu.VMEM((KC,), _i32),
            pltpu.VMEM((KC,), _i32),
            pltpu.VMEM_SHARED((NR, D), _f32),
            pltpu.SemaphoreType.DMA,
        ],
    )
    def k(h_hbm, src_hbm, dst_hbm, out_hbm, rows, src_v, dst_v, acc_sh, sem1):
        c = lax.axis_index("c")
        s = lax.axis_index("s")
        wid = c * NS + s
        wbase = wid * EPW

        def zero_body(i, _):
            for j in range(8):
                rows[i, pl.ds(j * 16, 16)] = jnp.zeros((16,), _f32)
            return 0
        lax.fori_loop(0, KC, zero_body, 0)
        stripe = s * STRIPE
        for r in range(STRIPE // KC):
            pltpu.sync_copy(rows, acc_sh.at[pl.ds(stripe + r * KC, KC)])
        plsc.subcore_barrier()

        def chunk(ci, _):
            base = wbase + ci * KC
            pltpu.sync_copy(src_hbm.at[pl.ds(base, KC)], src_v)
            pltpu.sync_copy(dst_hbm.at[pl.ds(base, KC)], dst_v)
            pltpu.async_copy(h_hbm.at[src_v], rows, sem1).wait()
            pltpu.sync_copy(rows, acc_sh.at[dst_v], add=True)
            return 0
        lax.fori_loop(0, NCHC, chunk, 0)

        plsc.subcore_barrier()
        pltpu.sync_copy(acc_sh.at[pl.ds(stripe, STRIPE)],
                        out_hbm.at[c, pl.ds(stripe, STRIPE)])

    return k(h, src, dst)


def _tc_tail(h_gat, gin_parts, x_res, batch2d, W1, b1, g1, bb1, lng, lnb,
             Wres, bres, Wg1, bg1, Wg2, bg2, Wh1, bh1, gh, bh, Wh2, bh2):
    def body(hg_r, gp_r, xres_r, batch_r, w1_r, b1_r, g1_r, bb1_r, lng_r,
             lnb_r, wres_r, bres_r, wg1_r, bg1_r, wg2_r, bg2_r, wh1_r, bh1_r,
             gh_r, bh_r, wh2_r, bh2_r, out_r):
        h = hg_r[...] + gp_r[0, :N, :] + gp_r[1, :N, :]
        h = _mmT(h, w1_r[...]) + b1_r[...]
        h = jnp.maximum(h, 0.0)
        h = h * g1_r[...] + bb1_r[...]
        h = h - jnp.mean(h)
        hc = h - jnp.mean(h)
        std = jnp.sqrt(jnp.mean(hc * hc))
        h = h / (std + 1e-5)
        h = h * lng_r[...] + lnb_r[...]
        h = h + _mmT(xres_r[...], wres_r[...]) + bres_r[...]
        h = jnp.maximum(h, 0.2 * h)
        t = jnp.tanh(_mmT(h, wg1_r[...]) + bg1_r[...])
        gate = jnp.sum(t * wg2_r[...], axis=1, keepdims=True) + bg2_r[0]
        ge = jnp.exp(gate)
        onehot = (batch_r[...]
                  == lax.broadcasted_iota(_i32, (G, 1), 0)).astype(_f32)
        gden = lax.dot_general(onehot, ge, (((1,), (0,)), ((), ())),
                               preferred_element_type=_f32,
                               precision=lax.Precision.HIGHEST)
        u = lax.dot_general(onehot, ge * h, (((1,), (0,)), ((), ())),
                            preferred_element_type=_f32,
                            precision=lax.Precision.HIGHEST)
        emb = u / (gden + 1e-16)
        outs = []
        for o in range(O):
            z = _mmT(emb, wh1_r[o]) + bh1_r[o]
            z = z * jax.nn.sigmoid(z)
            z = z * gh_r[o] + bh_r[o]
            outs.append(jnp.sum(z * wh2_r[o], axis=1, keepdims=True)
                        + bh2_r[o, 0])
        out_r[...] = jnp.concatenate(outs, axis=1)

    return pl.pallas_call(
        body,
        out_shape=jax.ShapeDtypeStruct((G, O), _f32),
    )(h_gat, gin_parts, x_res, batch2d, W1, b1, g1, bb1, lng, lnb,
      Wres, bres, Wg1, bg1, Wg2, bg2, Wh1, bh1, gh, bh, Wh2, bh2)


def kernel(x, edge_index, batch, Wp, bp, g0, b0, Wl, bl, Wr, br, att, bgat,
           W1, b1, g1, bb1, lng, lnb, Wres, bres, Wg1, bg1, Wg2, bg2,
           Wh1, bh1, gh, bh, Wh2, bh2):
    src = edge_index[0]
    dst = edge_index[1]
    h0, xl, xr = _tc_proj(x, Wp, bp, g0, b0, Wl, bl, Wr, br)
    ex, den_parts = _sc_scores(xl, xr, src, dst, att.reshape(H * DH))
    rden = _tc_rden(den_parts)
    gat_parts = _sc_gat_agg(xl, src, dst, ex, rden)
    h_gat = _tc_combine(gat_parts, bgat)
    gin_parts = _sc_gin_agg(h_gat, src, dst)
    out = _tc_tail(h_gat, gin_parts, h0, batch.reshape(1, N), W1, b1, g1,
                   bb1, lng, lnb, Wres, bres, Wg1, bg1, Wg2, bg2,
                   Wh1, bh1, gh, bh, Wh2, bh2)
    return out.reshape(G, O, 1)


# speedup vs baseline: 28.5445x; 1.1133x over previous
import functools

import jax
import jax.numpy as jnp
from jax import lax
from jax.experimental import pallas as pl
from jax.experimental.pallas import tpu as pltpu
from jax.experimental.pallas import tpu_sc as plsc

N = 10000
E = 320000
G = 64
D = 128
H = 4
DH = 32
O = 8

NC = 2
NS = 16
NW = NC * NS
EPW = E // NW
KA = 80
NCHA = EPW // KA
KB = 40
NCHB = EPW // KB
KC = 80
NCHC = EPW // KC
NR = 10240
STRIPE = NR // NS

_f32 = jnp.float32
_i32 = jnp.int32
_mesh = plsc.VectorSubcoreMesh(core_axis_name="c", subcore_axis_name="s")
_sc_params = pltpu.CompilerParams(needs_layout_passes=False)


def _mmT(a, w):
    return lax.dot_general(a, w, (((1,), (1,)), ((), ())),
                           preferred_element_type=jnp.float32,
                           precision=lax.Precision.HIGHEST)


def _tc_proj(x, Wp, bp, g0, b0, Wl, bl, Wr, br):
    BLK = 2000

    def body(x_r, wp_r, bp_r, g0_r, b0_r, wl_r, bl_r, wr_r, br_r,
             h0_r, xl_r, xr_r):
        h = _mmT(x_r[...], wp_r[...]) + bp_r[...]
        h = jnp.maximum(h, 0.0)
        h = h * g0_r[...] + b0_r[...]
        h0_r[...] = h
        xl_r[...] = _mmT(h, wl_r[...]) + bl_r[...]
        xr_r[...] = _mmT(h, wr_r[...]) + br_r[...]

    row_spec = pl.BlockSpec((BLK, D), lambda i: (i, 0))
    mat_spec = pl.BlockSpec((D, D), lambda i: (0, 0))
    vec_spec = pl.BlockSpec((D,), lambda i: (0,))
    return pl.pallas_call(
        body,
        grid=(N // BLK,),
        in_specs=[row_spec, mat_spec, vec_spec, vec_spec, vec_spec,
                  mat_spec, vec_spec, mat_spec, vec_spec],
        out_specs=[row_spec, row_spec, row_spec],
        out_shape=[jax.ShapeDtypeStruct((N, D), _f32)] * 3,
    )(x, Wp, bp, g0, b0, Wl, bl, Wr, br)


def _sc_scores(xl, xr, src, dst, att_flat):
    @functools.partial(
        pl.kernel,
        mesh=_mesh,
        compiler_params=_sc_params,
        out_type=[
            jax.ShapeDtypeStruct((E * H,), _f32),
            jax.ShapeDtypeStruct((NW, H * N), _f32),
        ],
        scratch_types=[
            pltpu.VMEM((KA, D), _f32),
            pltpu.VMEM((KA, D), _f32),
            pltpu.VMEM((KA,), _i32),
            pltpu.VMEM((KA,), _i32),
            pltpu.VMEM((D,), _f32),
            pltpu.VMEM((EPW * H,), _f32),
            pltpu.VMEM((H * N,), _f32),
            pltpu.SemaphoreType.DMA,
            pltpu.SemaphoreType.DMA,
        ],
    )
    def k(xl_hbm, xr_hbm, src_hbm, dst_hbm, att_hbm, ex_hbm, den_hbm,
          xl_rows, xr_rows, src_v, dst_v, att_v, ex_v, den_local,
          sem1, sem2):
        c = lax.axis_index("c")
        s = lax.axis_index("s")
        wid = c * NS + s
        wbase = wid * EPW

        def zero_body(i, _):
            den_local[pl.ds(i * 16, 16)] = jnp.zeros((16,), _f32)
            return 0
        lax.fori_loop(0, (H * N) // 16, zero_body, 0)

        pltpu.sync_copy(att_hbm, att_v)
        ii = lax.iota(_i32, 16)
        p8, p4, p2, p1 = ii ^ 8, ii ^ 4, ii ^ 2, ii ^ 1
        attv = [att_v[pl.ds(j * 16, 16)] for j in range(8)]
        lmask = [ii == t for t in range(16)]
        zero16 = jnp.zeros((16,), _f32)

        def chunk(ci, _):
            base = wbase + ci * KA
            pltpu.sync_copy(src_hbm.at[pl.ds(base, KA)], src_v)
            pltpu.sync_copy(dst_hbm.at[pl.ds(base, KA)], dst_v)
            cp1 = pltpu.async_copy(xl_hbm.at[src_v], xl_rows, sem1)
            cp2 = pltpu.async_copy(xr_hbm.at[dst_v], xr_rows, sem2)
            cp1.wait()
            cp2.wait()

            def grp(g, _):
                dst16 = dst_v[pl.ds(g * 16, 16)]
                exs = [zero16] * H
                for t in range(16):
                    e = g * 16 + t
                    for h in range(H):
                        j0 = h * 32
                        a = (xl_rows[e, pl.ds(j0, 16)]
                             + xr_rows[e, pl.ds(j0, 16)])
                        b = (xl_rows[e, pl.ds(j0 + 16, 16)]
                             + xr_rows[e, pl.ds(j0 + 16, 16)])
                        a = jnp.maximum(a, 0.2 * a) * attv[2 * h]
                        b = jnp.maximum(b, 0.2 * b) * attv[2 * h + 1]
                        v = a + b
                        v = v + v[p8]
                        v = v + v[p4]
                        v = v + v[p2]
                        v = v + v[p1]
                        exs[h] = jnp.where(lmask[t], v, exs[h])
                for h in range(H):
                    ex16 = jnp.exp(exs[h])
                    plsc.store_scatter(
                        ex_v, [(ci * KA + g * 16 + ii) * H + h], ex16)
                    plsc.addupdate_scatter(den_local, [dst16 + h * N], ex16)
                return 0
            lax.fori_loop(0, KA // 16, grp, 0)
            return 0
        lax.fori_loop(0, NCHA, chunk, 0)

        pltpu.sync_copy(ex_v, ex_hbm.at[pl.ds(wbase * H, EPW * H)])
        pltpu.sync_copy(den_local, den_hbm.at[wid])

    return k(xl, xr, src, dst, att_flat)


def _tc_rden(den_partials):
    def body(d_r, out_r):
        out_r[...] = 1.0 / (jnp.sum(d_r[...], axis=0) + 1e-16)

    return pl.pallas_call(
        body,
        out_shape=jax.ShapeDtypeStruct((H * N,), _f32),
    )(den_partials)


def _sc_gat_agg(xl, src, dst, ex, rden):
    @functools.partial(
        pl.kernel,
        mesh=_mesh,
        compiler_params=_sc_params,
        out_type=jax.ShapeDtypeStruct((NC, NR, D), _f32),
        scratch_types=[
            pltpu.VMEM((KB, D), _f32),
            pltpu.VMEM((KB,), _i32),
            pltpu.VMEM((KB,), _i32),
            pltpu.VMEM((KB * H,), _f32),
            pltpu.VMEM((KB * H,), _f32),
            pltpu.VMEM((H * N,), _f32),
            pltpu.VMEM_SHARED((NR, D), _f32),
            pltpu.SemaphoreType.DMA,
        ],
    )
    def k(xl_hbm, src_hbm, dst_hbm, ex_hbm, rden_hbm, out_hbm,
          rows, src_v, dst_v, ex_c, alpha_c, rden_v, acc_sh, sem1):
        c = lax.axis_index("c")
        s = lax.axis_index("s")
        wid = c * NS + s
        wbase = wid * EPW

        pltpu.sync_copy(rden_hbm, rden_v)

        def zero_body(i, _):
            for j in range(8):
                rows[i, pl.ds(j * 16, 16)] = jnp.zeros((16,), _f32)
            return 0
        lax.fori_loop(0, KB, zero_body, 0)
        stripe = s * STRIPE
        for r in range(STRIPE // KB):
            pltpu.sync_copy(rows, acc_sh.at[pl.ds(stripe + r * KB, KB)])
        plsc.subcore_barrier()

        ii = lax.iota(_i32, 16)
        erep = ii // H
        hrep = ii - erep * H

        def chunk(ci, _):
            base = wbase + ci * KB
            pltpu.sync_copy(src_hbm.at[pl.ds(base, KB)], src_v)
            pltpu.sync_copy(dst_hbm.at[pl.ds(base, KB)], dst_v)
            cp = pltpu.async_copy(xl_hbm.at[src_v], rows, sem1)
            pltpu.sync_copy(ex_hbm.at[pl.ds(base * H, KB * H)], ex_c)
            cp.wait()

            def grp(g, _):
                dst4 = plsc.load_gather(dst_v, [g * 4 + erep])
                rv = plsc.load_gather(rden_v, [dst4 + hrep * N])
                alpha_c[pl.ds(g * 16, 16)] = ex_c[pl.ds(g * 16, 16)] * rv
                return 0
            lax.fori_loop(0, (KB * H) // 16, grp, 0)

            def quad(q, _):
                av16 = alpha_c[pl.ds(q * 16, 16)]
                for t in range(4):
                    e = q * 4 + t
                    for h in range(H):
                        av = jnp.full((16,), av16[t * H + h], _f32)
                        j0 = h * 32
                        rows[e, pl.ds(j0, 16)] = rows[e, pl.ds(j0, 16)] * av
                        rows[e, pl.ds(j0 + 16, 16)] = (
                            rows[e, pl.ds(j0 + 16, 16)] * av)
                return 0
            lax.fori_loop(0, KB // 4, quad, 0)

            pltpu.sync_copy(rows, acc_sh.at[dst_v], add=True)
            return 0
        lax.fori_loop(0, NCHB, chunk, 0)

        plsc.subcore_barrier()
        pltpu.sync_copy(acc_sh.at[pl.ds(stripe, STRIPE)],
                        out_hbm.at[c, pl.ds(stripe, STRIPE)])

    return k(xl, src, dst, ex, rden)


def _tc_combine(parts, bias):
    def body(p_r, b_r, out_r):
        out_r[...] = p_r[0, :N, :] + p_r[1, :N, :] + b_r[...]

    return pl.pallas_call(
        body,
        out_shape=jax.ShapeDtypeStruct((N, D), _f32),
    )(parts, bias)


def _sc_gin_agg(h, src, dst):
    @functools.partial(
        pl.kernel,
        mesh=_mesh,
        compiler_params=_sc_params,
        out_type=jax.ShapeDtypeStruct((NC, NR, D), _f32),
        scratch_types=[
            pltpu.VMEM((KC, D), _f32),
            pltpu.VMEM((KC,), _i32),
            pltpu.VMEM((KC,), _i32),
            pltpu.VMEM_SHARED((NR, D), _f32),
            pltpu.SemaphoreType.DMA,
        ],
    )
    def k(h_hbm, src_hbm, dst_hbm, out_hbm, rows, src_v, dst_v, acc_sh, sem1):
        c = lax.axis_index("c")
        s = lax.axis_index("s")
        wid = c * NS + s
        wbase = wid * EPW

        def zero_body(i, _):
            for j in range(8):
                rows[i, pl.ds(j * 16, 16)] = jnp.zeros((16,), _f32)
            return 0
        lax.fori_loop(0, KC, zero_body, 0)
        stripe = s * STRIPE
        for r in range(STRIPE // KC):
            pltpu.sync_copy(rows, acc_sh.at[pl.ds(stripe + r * KC, KC)])
        plsc.subcore_barrier()

        def chunk(ci, _):
            base = wbase + ci * KC
            pltpu.sync_copy(src_hbm.at[pl.ds(base, KC)], src_v)
            pltpu.sync_copy(dst_hbm.at[pl.ds(base, KC)], dst_v)
            pltpu.async_copy(h_hbm.at[src_v], rows, sem1).wait()
            pltpu.sync_copy(rows, acc_sh.at[dst_v], add=True)
            return 0
        lax.fori_loop(0, NCHC, chunk, 0)

        plsc.subcore_barrier()
        pltpu.sync_copy(acc_sh.at[pl.ds(stripe, STRIPE)],
                        out_hbm.at[c, pl.ds(stripe, STRIPE)])

    return k(h, src, dst)


def _tc_tail(h_gat, gin_parts, x_res, batch2d, W1, b1, g1, bb1, lng, lnb,
             Wres, bres, Wg1, bg1, Wg2, bg2, Wh1, bh1, gh, bh, Wh2, bh2):
    def body(hg_r, gp_r, xres_r, batch_r, w1_r, b1_r, g1_r, bb1_r, lng_r,
             lnb_r, wres_r, bres_r, wg1_r, bg1_r, wg2_r, bg2_r, wh1_r, bh1_r,
             gh_r, bh_r, wh2_r, bh2_r, out_r):
        h = hg_r[...] + gp_r[0, :N, :] + gp_r[1, :N, :]
        h = _mmT(h, w1_r[...]) + b1_r[...]
        h = jnp.maximum(h, 0.0)
        h = h * g1_r[...] + bb1_r[...]
        h = h - jnp.mean(h)
        hc = h - jnp.mean(h)
        std = jnp.sqrt(jnp.mean(hc * hc))
        h = h / (std + 1e-5)
        h = h * lng_r[...] + lnb_r[...]
        h = h + _mmT(xres_r[...], wres_r[...]) + bres_r[...]
        h = jnp.maximum(h, 0.2 * h)
        t = jnp.tanh(_mmT(h, wg1_r[...]) + bg1_r[...])
        gate = jnp.sum(t * wg2_r[...], axis=1, keepdims=True) + bg2_r[0]
        ge = jnp.exp(gate)
        onehot = (batch_r[...]
                  == lax.broadcasted_iota(_i32, (G, 1), 0)).astype(_f32)
        gden = lax.dot_general(onehot, ge, (((1,), (0,)), ((), ())),
                               preferred_element_type=_f32,
                               precision=lax.Precision.HIGHEST)
        u = lax.dot_general(onehot, ge * h, (((1,), (0,)), ((), ())),
                            preferred_element_type=_f32,
                            precision=lax.Precision.HIGHEST)
        emb = u / (gden + 1e-16)
        outs = []
        for o in range(O):
            z = _mmT(emb, wh1_r[o]) + bh1_r[o]
            z = z * jax.nn.sigmoid(z)
            z = z * gh_r[o] + bh_r[o]
            outs.append(jnp.sum(z * wh2_r[o], axis=1, keepdims=True)
                        + bh2_r[o, 0])
        out_r[...] = jnp.concatenate(outs, axis=1)

    return pl.pallas_call(
        body,
        out_shape=jax.ShapeDtypeStruct((G, O), _f32),
    )(h_gat, gin_parts, x_res, batch2d, W1, b1, g1, bb1, lng, lnb,
      Wres, bres, Wg1, bg1, Wg2, bg2, Wh1, bh1, gh, bh, Wh2, bh2)


def kernel(x, edge_index, batch, Wp, bp, g0, b0, Wl, bl, Wr, br, att, bgat,
           W1, b1, g1, bb1, lng, lnb, Wres, bres, Wg1, bg1, Wg2, bg2,
           Wh1, bh1, gh, bh, Wh2, bh2):
    src = edge_index[0]
    dst = edge_index[1]
    h0, xl, xr = _tc_proj(x, Wp, bp, g0, b0, Wl, bl, Wr, br)
    ex, den_parts = _sc_scores(xl, xr, src, dst, att.reshape(H * DH))
    rden = _tc_rden(den_parts)
    gat_parts = _sc_gat_agg(xl, src, dst, ex, rden)
    h_gat = _tc_combine(gat_parts, bgat)
    gin_parts = _sc_gin_agg(h_gat, src, dst)
    out = _tc_tail(h_gat, gin_parts, h0, batch.reshape(1, N), W1, b1, g1,
                   bb1, lng, lnb, Wres, bres, Wg1, bg1, Wg2, bg2,
                   Wh1, bh1, gh, bh, Wh2, bh2)
    return out.reshape(G, O, 1)


# speedup vs baseline: 33.6882x; 1.1802x over previous
import functools

import jax
import jax.numpy as jnp
from jax import lax
from jax.experimental import pallas as pl
from jax.experimental.pallas import tpu as pltpu
from jax.experimental.pallas import tpu_sc as plsc

N = 10000
E = 320000
G = 64
D = 128
H = 4
DH = 32
O = 8

NC = 2
NS = 16
NW = NC * NS
EPW = E // NW
KA = 80
NCHA = EPW // KA
KB = 40
NCHB = EPW // KB
KC = 80
NCHC = EPW // KC
NR = 10240
STRIPE = NR // NS

_f32 = jnp.float32
_i32 = jnp.int32
_mesh = plsc.VectorSubcoreMesh(core_axis_name="c", subcore_axis_name="s")
_sc_params = pltpu.CompilerParams(needs_layout_passes=False)


def _mmT(a, w):
    return lax.dot_general(a, w, (((1,), (1,)), ((), ())),
                           preferred_element_type=jnp.float32,
                           precision=lax.Precision.HIGHEST)


def _tc_proj(x, Wp, bp, g0, b0, Wl, bl, Wr, br):
    BLK = 2000

    def body(x_r, wp_r, bp_r, g0_r, b0_r, wl_r, bl_r, wr_r, br_r,
             h0_r, xl_r, xr_r):
        h = _mmT(x_r[...], wp_r[...]) + bp_r[...]
        h = jnp.maximum(h, 0.0)
        h = h * g0_r[...] + b0_r[...]
        h0_r[...] = h
        xl_r[...] = _mmT(h, wl_r[...]) + bl_r[...]
        xr_r[...] = _mmT(h, wr_r[...]) + br_r[...]

    row_spec = pl.BlockSpec((BLK, D), lambda i: (i, 0))
    mat_spec = pl.BlockSpec((D, D), lambda i: (0, 0))
    vec_spec = pl.BlockSpec((D,), lambda i: (0,))
    return pl.pallas_call(
        body,
        grid=(N // BLK,),
        in_specs=[row_spec, mat_spec, vec_spec, vec_spec, vec_spec,
                  mat_spec, vec_spec, mat_spec, vec_spec],
        out_specs=[row_spec, row_spec, row_spec],
        out_shape=[jax.ShapeDtypeStruct((N, D), _f32)] * 3,
    )(x, Wp, bp, g0, b0, Wl, bl, Wr, br)


def _sc_scores(xl, xr, src, dst, att_flat):
    @functools.partial(
        pl.kernel,
        mesh=_mesh,
        compiler_params=_sc_params,
        out_type=[
            jax.ShapeDtypeStruct((E * H,), _f32),
            jax.ShapeDtypeStruct((NW, H * N), _f32),
        ],
        scratch_types=[
            pltpu.VMEM((KA, D), _f32),
            pltpu.VMEM((KA, D), _f32),
            pltpu.VMEM((KA, D), _f32),
            pltpu.VMEM((KA, D), _f32),
            pltpu.VMEM((KA,), _i32),
            pltpu.VMEM((KA,), _i32),
            pltpu.VMEM((KA,), _i32),
            pltpu.VMEM((KA,), _i32),
            pltpu.VMEM((D,), _f32),
            pltpu.VMEM((EPW * H,), _f32),
            pltpu.VMEM((H * N,), _f32),
            pltpu.SemaphoreType.DMA,
            pltpu.SemaphoreType.DMA,
            pltpu.SemaphoreType.DMA,
            pltpu.SemaphoreType.DMA,
        ],
    )
    def k(xl_hbm, xr_hbm, src_hbm, dst_hbm, att_hbm, ex_hbm, den_hbm,
          xl_rows, xr_rows, xl_rows1, xr_rows1, src_v, dst_v, src_v1, dst_v1,
          att_v, ex_v, den_local, sem1, sem2, sem3, sem4):
        c = lax.axis_index("c")
        s = lax.axis_index("s")
        wid = c * NS + s
        wbase = wid * EPW

        def zero_body(i, _):
            den_local[pl.ds(i * 16, 16)] = jnp.zeros((16,), _f32)
            return 0
        lax.fori_loop(0, (H * N) // 16, zero_body, 0)

        pltpu.sync_copy(att_hbm, att_v)
        ii = lax.iota(_i32, 16)
        p8, p4, p2, p1 = ii ^ 8, ii ^ 4, ii ^ 2, ii ^ 1
        attv = [att_v[pl.ds(j * 16, 16)] for j in range(8)]
        lmask = [ii == t for t in range(16)]
        zero16 = jnp.zeros((16,), _f32)

        def start(ci, xl_b, xr_b, sv, dv, sa, sb):
            base = wbase + ci * KA
            pltpu.sync_copy(src_hbm.at[pl.ds(base, KA)], sv)
            pltpu.sync_copy(dst_hbm.at[pl.ds(base, KA)], dv)
            return (pltpu.async_copy(xl_hbm.at[sv], xl_b, sa),
                    pltpu.async_copy(xr_hbm.at[dv], xr_b, sb))

        def compute(ci, xl_b, xr_b, dv):
            def grp(g, _):
                dst16 = dv[pl.ds(g * 16, 16)]
                exs = [zero16] * H
                for t in range(16):
                    e = g * 16 + t
                    for h in range(H):
                        j0 = h * 32
                        a = xl_b[e, pl.ds(j0, 16)] + xr_b[e, pl.ds(j0, 16)]
                        b = (xl_b[e, pl.ds(j0 + 16, 16)]
                             + xr_b[e, pl.ds(j0 + 16, 16)])
                        a = jnp.maximum(a, 0.2 * a) * attv[2 * h]
                        b = jnp.maximum(b, 0.2 * b) * attv[2 * h + 1]
                        v = a + b
                        v = v + v[p8]
                        v = v + v[p4]
                        v = v + v[p2]
                        v = v + v[p1]
                        exs[h] = jnp.where(lmask[t], v, exs[h])
                for h in range(H):
                    ex16 = jnp.exp(exs[h])
                    plsc.store_scatter(
                        ex_v, [(ci * KA + g * 16 + ii) * H + h], ex16)
                    plsc.addupdate_scatter(den_local, [dst16 + h * N], ex16)
                return 0
            lax.fori_loop(0, KA // 16, grp, 0)

        def pair(p, _):
            c0 = 2 * p
            c1 = c0 + 1
            g0a, g0b = start(c0, xl_rows, xr_rows, src_v, dst_v, sem1, sem2)
            g1a, g1b = start(c1, xl_rows1, xr_rows1, src_v1, dst_v1,
                             sem3, sem4)
            g0a.wait()
            g0b.wait()
            compute(c0, xl_rows, xr_rows, dst_v)
            g1a.wait()
            g1b.wait()
            compute(c1, xl_rows1, xr_rows1, dst_v1)
            return 0
        lax.fori_loop(0, NCHA // 2, pair, 0)
        ga, gb = start(NCHA - 1, xl_rows, xr_rows, src_v, dst_v, sem1, sem2)
        ga.wait()
        gb.wait()
        compute(NCHA - 1, xl_rows, xr_rows, dst_v)

        pltpu.sync_copy(ex_v, ex_hbm.at[pl.ds(wbase * H, EPW * H)])
        pltpu.sync_copy(den_local, den_hbm.at[wid])

    return k(xl, xr, src, dst, att_flat)


def _tc_rden(den_partials):
    def body(d_r, out_r):
        out_r[...] = 1.0 / (jnp.sum(d_r[...], axis=0) + 1e-16)

    return pl.pallas_call(
        body,
        out_shape=jax.ShapeDtypeStruct((H * N,), _f32),
    )(den_partials)


def _sc_gat_agg(xl, src, dst, ex, rden):
    @functools.partial(
        pl.kernel,
        mesh=_mesh,
        compiler_params=_sc_params,
        out_type=jax.ShapeDtypeStruct((NC, NR, D), _f32),
        scratch_types=[
            pltpu.VMEM((KB, D), _f32),
            pltpu.VMEM((KB,), _i32),
            pltpu.VMEM((KB,), _i32),
            pltpu.VMEM((KB * H,), _f32),
            pltpu.VMEM((KB * H,), _f32),
            pltpu.VMEM((H * N,), _f32),
            pltpu.VMEM_SHARED((NR, D), _f32),
            pltpu.SemaphoreType.DMA,
        ],
    )
    def k(xl_hbm, src_hbm, dst_hbm, ex_hbm, rden_hbm, out_hbm,
          rows, src_v, dst_v, ex_c, alpha_c, rden_v, acc_sh, sem1):
        c = lax.axis_index("c")
        s = lax.axis_index("s")
        wid = c * NS + s
        wbase = wid * EPW

        pltpu.sync_copy(rden_hbm, rden_v)

        def zero_body(i, _):
            for j in range(8):
                rows[i, pl.ds(j * 16, 16)] = jnp.zeros((16,), _f32)
            return 0
        lax.fori_loop(0, KB, zero_body, 0)
        stripe = s * STRIPE
        for r in range(STRIPE // KB):
            pltpu.sync_copy(rows, acc_sh.at[pl.ds(stripe + r * KB, KB)])
        plsc.subcore_barrier()

        ii = lax.iota(_i32, 16)
        erep = ii // H
        hrep = ii - erep * H

        def chunk(ci, _):
            base = wbase + ci * KB
            pltpu.sync_copy(src_hbm.at[pl.ds(base, KB)], src_v)
            pltpu.sync_copy(dst_hbm.at[pl.ds(base, KB)], dst_v)
            cp = pltpu.async_copy(xl_hbm.at[src_v], rows, sem1)
            pltpu.sync_copy(ex_hbm.at[pl.ds(base * H, KB * H)], ex_c)
            cp.wait()

            def grp(g, _):
                dst4 = plsc.load_gather(dst_v, [g * 4 + erep])
                rv = plsc.load_gather(rden_v, [dst4 + hrep * N])
                alpha_c[pl.ds(g * 16, 16)] = ex_c[pl.ds(g * 16, 16)] * rv
                return 0
            lax.fori_loop(0, (KB * H) // 16, grp, 0)

            def quad(q, _):
                av16 = alpha_c[pl.ds(q * 16, 16)]
                for t in range(4):
                    e = q * 4 + t
                    for h in range(H):
                        av = jnp.full((16,), av16[t * H + h], _f32)
                        j0 = h * 32
                        rows[e, pl.ds(j0, 16)] = rows[e, pl.ds(j0, 16)] * av
                        rows[e, pl.ds(j0 + 16, 16)] = (
                            rows[e, pl.ds(j0 + 16, 16)] * av)
                return 0
            lax.fori_loop(0, KB // 4, quad, 0)

            pltpu.sync_copy(rows, acc_sh.at[dst_v], add=True)
            return 0
        lax.fori_loop(0, NCHB, chunk, 0)

        plsc.subcore_barrier()
        pltpu.sync_copy(acc_sh.at[pl.ds(stripe, STRIPE)],
                        out_hbm.at[c, pl.ds(stripe, STRIPE)])

    return k(xl, src, dst, ex, rden)


def _tc_combine(parts, bias):
    def body(p_r, b_r, out_r):
        out_r[...] = p_r[0, :N, :] + p_r[1, :N, :] + b_r[...]

    return pl.pallas_call(
        body,
        out_shape=jax.ShapeDtypeStruct((N, D), _f32),
    )(parts, bias)


def _sc_gin_agg(h, src, dst):
    @functools.partial(
        pl.kernel,
        mesh=_mesh,
        compiler_params=_sc_params,
        out_type=jax.ShapeDtypeStruct((NC, NR, D), _f32),
        scratch_types=[
            pltpu.VMEM((KC, D), _f32),
            pltpu.VMEM((KC, D), _f32),
            pltpu.VMEM((KC,), _i32),
            pltpu.VMEM((KC,), _i32),
            pltpu.VMEM((KC,), _i32),
            pltpu.VMEM((KC,), _i32),
            pltpu.VMEM_SHARED((NR, D), _f32),
            pltpu.SemaphoreType.DMA,
            pltpu.SemaphoreType.DMA,
            pltpu.SemaphoreType.DMA,
            pltpu.SemaphoreType.DMA,
        ],
    )
    def k(h_hbm, src_hbm, dst_hbm, out_hbm, rows, rows1, src_v, dst_v,
          src_v1, dst_v1, acc_sh, sem1, sem2, sem3, sem4):
        c = lax.axis_index("c")
        s = lax.axis_index("s")
        wid = c * NS + s
        wbase = wid * EPW

        def zero_body(i, _):
            for j in range(8):
                rows[i, pl.ds(j * 16, 16)] = jnp.zeros((16,), _f32)
            return 0
        lax.fori_loop(0, KC, zero_body, 0)
        stripe = s * STRIPE
        for r in range(STRIPE // KC):
            pltpu.sync_copy(rows, acc_sh.at[pl.ds(stripe + r * KC, KC)])
        plsc.subcore_barrier()

        def start(ci, rows_b, sv, dv, sem):
            base = wbase + ci * KC
            pltpu.sync_copy(src_hbm.at[pl.ds(base, KC)], sv)
            pltpu.sync_copy(dst_hbm.at[pl.ds(base, KC)], dv)
            return pltpu.async_copy(h_hbm.at[sv], rows_b, sem)

        def pair(p, _):
            c0 = 2 * p
            g0 = start(c0, rows, src_v, dst_v, sem1)
            g1 = start(c0 + 1, rows1, src_v1, dst_v1, sem2)
            g0.wait()
            s0 = pltpu.async_copy(rows, acc_sh.at[dst_v], sem3, add=True)
            g1.wait()
            s1 = pltpu.async_copy(rows1, acc_sh.at[dst_v1], sem4, add=True)
            s0.wait()
            s1.wait()
            return 0
        lax.fori_loop(0, NCHC // 2, pair, 0)
        gt = start(NCHC - 1, rows, src_v, dst_v, sem1)
        gt.wait()
        pltpu.sync_copy(rows, acc_sh.at[dst_v], add=True)

        plsc.subcore_barrier()
        pltpu.sync_copy(acc_sh.at[pl.ds(stripe, STRIPE)],
                        out_hbm.at[c, pl.ds(stripe, STRIPE)])

    return k(h, src, dst)


def _tc_tail(h_gat, gin_parts, x_res, batch2d, W1, b1, g1, bb1, lng, lnb,
             Wres, bres, Wg1, bg1, Wg2, bg2, Wh1, bh1, gh, bh, Wh2, bh2):
    def body(hg_r, gp_r, xres_r, batch_r, w1_r, b1_r, g1_r, bb1_r, lng_r,
             lnb_r, wres_r, bres_r, wg1_r, bg1_r, wg2_r, bg2_r, wh1_r, bh1_r,
             gh_r, bh_r, wh2_r, bh2_r, out_r):
        h = hg_r[...] + gp_r[0, :N, :] + gp_r[1, :N, :]
        h = _mmT(h, w1_r[...]) + b1_r[...]
        h = jnp.maximum(h, 0.0)
        h = h * g1_r[...] + bb1_r[...]
        h = h - jnp.mean(h)
        hc = h - jnp.mean(h)
        std = jnp.sqrt(jnp.mean(hc * hc))
        h = h / (std + 1e-5)
        h = h * lng_r[...] + lnb_r[...]
        h = h + _mmT(xres_r[...], wres_r[...]) + bres_r[...]
        h = jnp.maximum(h, 0.2 * h)
        t = jnp.tanh(_mmT(h, wg1_r[...]) + bg1_r[...])
        gate = jnp.sum(t * wg2_r[...], axis=1, keepdims=True) + bg2_r[0]
        ge = jnp.exp(gate)
        onehot = (batch_r[...]
                  == lax.broadcasted_iota(_i32, (G, 1), 0)).astype(_f32)
        gden = lax.dot_general(onehot, ge, (((1,), (0,)), ((), ())),
                               preferred_element_type=_f32,
                               precision=lax.Precision.HIGHEST)
        u = lax.dot_general(onehot, ge * h, (((1,), (0,)), ((), ())),
                            preferred_element_type=_f32,
                            precision=lax.Precision.HIGHEST)
        emb = u / (gden + 1e-16)
        outs = []
        for o in range(O):
            z = _mmT(emb, wh1_r[o]) + bh1_r[o]
            z = z * jax.nn.sigmoid(z)
            z = z * gh_r[o] + bh_r[o]
            outs.append(jnp.sum(z * wh2_r[o], axis=1, keepdims=True)
                        + bh2_r[o, 0])
        out_r[...] = jnp.concatenate(outs, axis=1)

    return pl.pallas_call(
        body,
        out_shape=jax.ShapeDtypeStruct((G, O), _f32),
    )(h_gat, gin_parts, x_res, batch2d, W1, b1, g1, bb1, lng, lnb,
      Wres, bres, Wg1, bg1, Wg2, bg2, Wh1, bh1, gh, bh, Wh2, bh2)


def kernel(x, edge_index, batch, Wp, bp, g0, b0, Wl, bl, Wr, br, att, bgat,
           W1, b1, g1, bb1, lng, lnb, Wres, bres, Wg1, bg1, Wg2, bg2,
           Wh1, bh1, gh, bh, Wh2, bh2):
    src = edge_index[0]
    dst = edge_index[1]
    h0, xl, xr = _tc_proj(x, Wp, bp, g0, b0, Wl, bl, Wr, br)
    ex, den_parts = _sc_scores(xl, xr, src, dst, att.reshape(H * DH))
    rden = _tc_rden(den_parts)
    gat_parts = _sc_gat_agg(xl, src, dst, ex, rden)
    h_gat = _tc_combine(gat_parts, bgat)
    gin_parts = _sc_gin_agg(h_gat, src, dst)
    out = _tc_tail(h_gat, gin_parts, h0, batch.reshape(1, N), W1, b1, g1,
                   bb1, lng, lnb, Wres, bres, Wg1, bg1, Wg2, bg2,
                   Wh1, bh1, gh, bh, Wh2, bh2)
    return out.reshape(G, O, 1)


# speedup vs baseline: 41.5649x; 1.2338x over previous
import functools

import jax
import jax.numpy as jnp
from jax import lax
from jax.experimental import pallas as pl
from jax.experimental.pallas import tpu as pltpu
from jax.experimental.pallas import tpu_sc as plsc

N = 10000
E = 320000
G = 64
D = 128
H = 4
DH = 32
O = 8

NC = 2
NS = 16
NW = NC * NS
EPW = E // NW
KA = 80
NCHA = EPW // KA
KB1 = 2000
KB = 80
NCHB = EPW // KB
KC = 80
NCHC = EPW // KC
NR = 10240
STRIPE = NR // NS

_f32 = jnp.float32
_i32 = jnp.int32
_mesh = plsc.VectorSubcoreMesh(core_axis_name="c", subcore_axis_name="s")
_sc_params = pltpu.CompilerParams(needs_layout_passes=False)


def _mmT(a, w):
    return lax.dot_general(a, w, (((1,), (1,)), ((), ())),
                           preferred_element_type=jnp.float32,
                           precision=lax.Precision.HIGHEST)


def _tc_proj(x, Wp, bp, g0, b0, Wl, bl, Wr, br):
    BLK = 2000

    def body(x_r, wp_r, bp_r, g0_r, b0_r, wl_r, bl_r, wr_r, br_r,
             h0_r, xl_r, xr_r):
        h = _mmT(x_r[...], wp_r[...]) + bp_r[...]
        h = jnp.maximum(h, 0.0)
        h = h * g0_r[...] + b0_r[...]
        h0_r[...] = h
        xl_r[...] = _mmT(h, wl_r[...]) + bl_r[...]
        xr_r[...] = _mmT(h, wr_r[...]) + br_r[...]

    row_spec = pl.BlockSpec((BLK, D), lambda i: (i, 0))
    mat_spec = pl.BlockSpec((D, D), lambda i: (0, 0))
    vec_spec = pl.BlockSpec((D,), lambda i: (0,))
    return pl.pallas_call(
        body,
        grid=(N // BLK,),
        in_specs=[row_spec, mat_spec, vec_spec, vec_spec, vec_spec,
                  mat_spec, vec_spec, mat_spec, vec_spec],
        out_specs=[row_spec, row_spec, row_spec],
        out_shape=[jax.ShapeDtypeStruct((N, D), _f32)] * 3,
    )(x, Wp, bp, g0, b0, Wl, bl, Wr, br)


def _sc_scores(xl, xr, src, dst, att_flat):
    @functools.partial(
        pl.kernel,
        mesh=_mesh,
        compiler_params=_sc_params,
        out_type=[
            jax.ShapeDtypeStruct((E * H,), _f32),
            jax.ShapeDtypeStruct((NW, H * N), _f32),
        ],
        scratch_types=[
            pltpu.VMEM((KA, D), _f32),
            pltpu.VMEM((KA, D), _f32),
            pltpu.VMEM((KA, D), _f32),
            pltpu.VMEM((KA, D), _f32),
            pltpu.VMEM((KA,), _i32),
            pltpu.VMEM((KA,), _i32),
            pltpu.VMEM((KA,), _i32),
            pltpu.VMEM((KA,), _i32),
            pltpu.VMEM((D,), _f32),
            pltpu.VMEM((EPW * H,), _f32),
            pltpu.VMEM((H * N,), _f32),
            pltpu.SemaphoreType.DMA,
            pltpu.SemaphoreType.DMA,
            pltpu.SemaphoreType.DMA,
            pltpu.SemaphoreType.DMA,
        ],
    )
    def k(xl_hbm, xr_hbm, src_hbm, dst_hbm, att_hbm, ex_hbm, den_hbm,
          xl_rows, xr_rows, xl_rows1, xr_rows1, src_v, dst_v, src_v1, dst_v1,
          att_v, ex_v, den_local, sem1, sem2, sem3, sem4):
        c = lax.axis_index("c")
        s = lax.axis_index("s")
        wid = c * NS + s
        wbase = wid * EPW

        def zero_body(i, _):
            den_local[pl.ds(i * 16, 16)] = jnp.zeros((16,), _f32)
            return 0
        lax.fori_loop(0, (H * N) // 16, zero_body, 0)

        pltpu.sync_copy(att_hbm, att_v)
        ii = lax.iota(_i32, 16)
        p8, p4, p2, p1 = ii ^ 8, ii ^ 4, ii ^ 2, ii ^ 1
        attv = [att_v[pl.ds(j * 16, 16)] for j in range(8)]
        lmask = [ii == t for t in range(16)]
        zero16 = jnp.zeros((16,), _f32)

        def start(ci, xl_b, xr_b, sv, dv, sa, sb):
            base = wbase + ci * KA
            pltpu.sync_copy(src_hbm.at[pl.ds(base, KA)], sv)
            pltpu.sync_copy(dst_hbm.at[pl.ds(base, KA)], dv)
            return (pltpu.async_copy(xl_hbm.at[sv], xl_b, sa),
                    pltpu.async_copy(xr_hbm.at[dv], xr_b, sb))

        def compute(ci, xl_b, xr_b, dv):
            def grp(g, _):
                dst16 = dv[pl.ds(g * 16, 16)]
                exs = [zero16] * H
                for t in range(16):
                    e = g * 16 + t
                    for h in range(H):
                        j0 = h * 32
                        a = xl_b[e, pl.ds(j0, 16)] + xr_b[e, pl.ds(j0, 16)]
                        b = (xl_b[e, pl.ds(j0 + 16, 16)]
                             + xr_b[e, pl.ds(j0 + 16, 16)])
                        a = jnp.maximum(a, 0.2 * a) * attv[2 * h]
                        b = jnp.maximum(b, 0.2 * b) * attv[2 * h + 1]
                        v = a + b
                        v = v + v[p8]
                        v = v + v[p4]
                        v = v + v[p2]
                        v = v + v[p1]
                        exs[h] = jnp.where(lmask[t], v, exs[h])
                for h in range(H):
                    ex16 = jnp.exp(exs[h])
                    plsc.store_scatter(
                        ex_v, [(ci * KA + g * 16 + ii) * H + h], ex16)
                    plsc.addupdate_scatter(den_local, [dst16 + h * N], ex16)
                return 0
            lax.fori_loop(0, KA // 16, grp, 0)

        def pair(p, _):
            c0 = 2 * p
            c1 = c0 + 1
            g0a, g0b = start(c0, xl_rows, xr_rows, src_v, dst_v, sem1, sem2)
            g1a, g1b = start(c1, xl_rows1, xr_rows1, src_v1, dst_v1,
                             sem3, sem4)
            g0a.wait()
            g0b.wait()
            compute(c0, xl_rows, xr_rows, dst_v)
            g1a.wait()
            g1b.wait()
            compute(c1, xl_rows1, xr_rows1, dst_v1)
            return 0
        lax.fori_loop(0, NCHA // 2, pair, 0)
        ga, gb = start(NCHA - 1, xl_rows, xr_rows, src_v, dst_v, sem1, sem2)
        ga.wait()
        gb.wait()
        compute(NCHA - 1, xl_rows, xr_rows, dst_v)

        pltpu.sync_copy(ex_v, ex_hbm.at[pl.ds(wbase * H, EPW * H)])
        pltpu.sync_copy(den_local, den_hbm.at[wid])

    return k(xl, xr, src, dst, att_flat)


def _tc_rden(den_partials):
    def body(d_r, out_r):
        out_r[...] = 1.0 / (jnp.sum(d_r[...], axis=0) + 1e-16)

    return pl.pallas_call(
        body,
        out_shape=jax.ShapeDtypeStruct((H * N,), _f32),
    )(den_partials)


def _sc_alpha(ex, rden, dst):
    @functools.partial(
        pl.kernel,
        mesh=_mesh,
        compiler_params=_sc_params,
        out_type=jax.ShapeDtypeStruct((E * H,), _f32),
        scratch_types=[
            pltpu.VMEM((H * N,), _f32),
            pltpu.VMEM((KB1 * H,), _f32),
            pltpu.VMEM((KB1 * H,), _f32),
            pltpu.VMEM((KB1,), _i32),
        ],
    )
    def k(ex_hbm, rden_hbm, dst_hbm, alpha_hbm, rden_v, ex_c, al_c, dst_c):
        c = lax.axis_index("c")
        s = lax.axis_index("s")
        wbase = (c * NS + s) * EPW
        pltpu.sync_copy(rden_hbm, rden_v)
        ii = lax.iota(_i32, 16)
        erep = ii // H
        hrep = ii - erep * H

        def chunk(ci, _):
            base = wbase + ci * KB1
            pltpu.sync_copy(dst_hbm.at[pl.ds(base, KB1)], dst_c)
            pltpu.sync_copy(ex_hbm.at[pl.ds(base * H, KB1 * H)], ex_c)

            def grp(g, _):
                dst4 = plsc.load_gather(dst_c, [g * 4 + erep])
                rv = plsc.load_gather(rden_v, [dst4 + hrep * N])
                al_c[pl.ds(g * 16, 16)] = ex_c[pl.ds(g * 16, 16)] * rv
                return 0
            lax.fori_loop(0, (KB1 * H) // 16, grp, 0)
            pltpu.sync_copy(al_c, alpha_hbm.at[pl.ds(base * H, KB1 * H)])
            return 0
        lax.fori_loop(0, EPW // KB1, chunk, 0)

    return k(ex, rden, dst)


def _sc_gat_agg(xl, src, dst, alpha):
    @functools.partial(
        pl.kernel,
        mesh=_mesh,
        compiler_params=_sc_params,
        out_type=jax.ShapeDtypeStruct((NC, NR, D), _f32),
        scratch_types=[
            pltpu.VMEM((KB, D), _f32),
            pltpu.VMEM((KB, D), _f32),
            pltpu.VMEM((KB * H,), _f32),
            pltpu.VMEM((KB * H,), _f32),
            pltpu.VMEM((KB,), _i32),
            pltpu.VMEM((KB,), _i32),
            pltpu.VMEM((KB,), _i32),
            pltpu.VMEM((KB,), _i32),
            pltpu.VMEM_SHARED((NR, D), _f32),
            pltpu.SemaphoreType.DMA,
            pltpu.SemaphoreType.DMA,
            pltpu.SemaphoreType.DMA,
            pltpu.SemaphoreType.DMA,
        ],
    )
    def k(xl_hbm, src_hbm, dst_hbm, alpha_hbm, out_hbm,
          rows, rows1, al_c0, al_c1, src_v, dst_v, src_v1, dst_v1,
          acc_sh, sem1, sem2, sem3, sem4):
        c = lax.axis_index("c")
        s = lax.axis_index("s")
        wbase = (c * NS + s) * EPW

        def zero_body(i, _):
            for j in range(8):
                rows[i, pl.ds(j * 16, 16)] = jnp.zeros((16,), _f32)
            return 0
        lax.fori_loop(0, KB, zero_body, 0)
        stripe = s * STRIPE
        for r in range(STRIPE // KB):
            pltpu.sync_copy(rows, acc_sh.at[pl.ds(stripe + r * KB, KB)])
        plsc.subcore_barrier()

        def start(ci, rows_b, al_b, sv, dv, sem):
            base = wbase + ci * KB
            pltpu.sync_copy(src_hbm.at[pl.ds(base, KB)], sv)
            pltpu.sync_copy(dst_hbm.at[pl.ds(base, KB)], dv)
            pltpu.sync_copy(alpha_hbm.at[pl.ds(base * H, KB * H)], al_b)
            return pltpu.async_copy(xl_hbm.at[sv], rows_b, sem)

        def scale(rows_b, al_b):
            def quad(q, _):
                av16 = al_b[pl.ds(q * 16, 16)]
                for t in range(4):
                    e = q * 4 + t
                    for h in range(H):
                        av = jnp.full((16,), av16[t * H + h], _f32)
                        j0 = h * 32
                        rows_b[e, pl.ds(j0, 16)] = (
                            rows_b[e, pl.ds(j0, 16)] * av)
                        rows_b[e, pl.ds(j0 + 16, 16)] = (
                            rows_b[e, pl.ds(j0 + 16, 16)] * av)
                return 0
            lax.fori_loop(0, KB // 4, quad, 0)

        def pair(p, _):
            c0 = 2 * p
            g0 = start(c0, rows, al_c0, src_v, dst_v, sem1)
            g1 = start(c0 + 1, rows1, al_c1, src_v1, dst_v1, sem2)
            g0.wait()
            scale(rows, al_c0)
            s0 = pltpu.async_copy(rows, acc_sh.at[dst_v], sem3, add=True)
            g1.wait()
            scale(rows1, al_c1)
            s1 = pltpu.async_copy(rows1, acc_sh.at[dst_v1], sem4, add=True)
            s0.wait()
            s1.wait()
            return 0
        lax.fori_loop(0, NCHB // 2, pair, 0)
        gt = start(NCHB - 1, rows, al_c0, src_v, dst_v, sem1)
        gt.wait()
        scale(rows, al_c0)
        pltpu.sync_copy(rows, acc_sh.at[dst_v], add=True)

        plsc.subcore_barrier()
        pltpu.sync_copy(acc_sh.at[pl.ds(stripe, STRIPE)],
                        out_hbm.at[c, pl.ds(stripe, STRIPE)])

    return k(xl, src, dst, alpha)


def _tc_combine(parts, bias):
    def body(p_r, b_r, out_r):
        out_r[...] = p_r[0, :N, :] + p_r[1, :N, :] + b_r[...]

    return pl.pallas_call(
        body,
        out_shape=jax.ShapeDtypeStruct((N, D), _f32),
    )(parts, bias)


def _sc_gin_agg(h, src, dst):
    @functools.partial(
        pl.kernel,
        mesh=_mesh,
        compiler_params=_sc_params,
        out_type=jax.ShapeDtypeStruct((NC, NR, D), _f32),
        scratch_types=[
            pltpu.VMEM((KC, D), _f32),
            pltpu.VMEM((KC, D), _f32),
            pltpu.VMEM((KC,), _i32),
            pltpu.VMEM((KC,), _i32),
            pltpu.VMEM((KC,), _i32),
            pltpu.VMEM((KC,), _i32),
            pltpu.VMEM_SHARED((NR, D), _f32),
            pltpu.SemaphoreType.DMA,
            pltpu.SemaphoreType.DMA,
            pltpu.SemaphoreType.DMA,
            pltpu.SemaphoreType.DMA,
        ],
    )
    def k(h_hbm, src_hbm, dst_hbm, out_hbm, rows, rows1, src_v, dst_v,
          src_v1, dst_v1, acc_sh, sem1, sem2, sem3, sem4):
        c = lax.axis_index("c")
        s = lax.axis_index("s")
        wid = c * NS + s
        wbase = wid * EPW

        def zero_body(i, _):
            for j in range(8):
                rows[i, pl.ds(j * 16, 16)] = jnp.zeros((16,), _f32)
            return 0
        lax.fori_loop(0, KC, zero_body, 0)
        stripe = s * STRIPE
        for r in range(STRIPE // KC):
            pltpu.sync_copy(rows, acc_sh.at[pl.ds(stripe + r * KC, KC)])
        plsc.subcore_barrier()

        def start(ci, rows_b, sv, dv, sem):
            base = wbase + ci * KC
            pltpu.sync_copy(src_hbm.at[pl.ds(base, KC)], sv)
            pltpu.sync_copy(dst_hbm.at[pl.ds(base, KC)], dv)
            return pltpu.async_copy(h_hbm.at[sv], rows_b, sem)

        def pair(p, _):
            c0 = 2 * p
            g0 = start(c0, rows, src_v, dst_v, sem1)
            g1 = start(c0 + 1, rows1, src_v1, dst_v1, sem2)
            g0.wait()
            s0 = pltpu.async_copy(rows, acc_sh.at[dst_v], sem3, add=True)
            g1.wait()
            s1 = pltpu.async_copy(rows1, acc_sh.at[dst_v1], sem4, add=True)
            s0.wait()
            s1.wait()
            return 0
        lax.fori_loop(0, NCHC // 2, pair, 0)
        gt = start(NCHC - 1, rows, src_v, dst_v, sem1)
        gt.wait()
        pltpu.sync_copy(rows, acc_sh.at[dst_v], add=True)

        plsc.subcore_barrier()
        pltpu.sync_copy(acc_sh.at[pl.ds(stripe, STRIPE)],
                        out_hbm.at[c, pl.ds(stripe, STRIPE)])

    return k(h, src, dst)


def _tc_tail(h_gat, gin_parts, x_res, batch2d, W1, b1, g1, bb1, lng, lnb,
             Wres, bres, Wg1, bg1, Wg2, bg2, Wh1, bh1, gh, bh, Wh2, bh2):
    def body(hg_r, gp_r, xres_r, batch_r, w1_r, b1_r, g1_r, bb1_r, lng_r,
             lnb_r, wres_r, bres_r, wg1_r, bg1_r, wg2_r, bg2_r, wh1_r, bh1_r,
             gh_r, bh_r, wh2_r, bh2_r, out_r):
        h = hg_r[...] + gp_r[0, :N, :] + gp_r[1, :N, :]
        h = _mmT(h, w1_r[...]) + b1_r[...]
        h = jnp.maximum(h, 0.0)
        h = h * g1_r[...] + bb1_r[...]
        h = h - jnp.mean(h)
        hc = h - jnp.mean(h)
        std = jnp.sqrt(jnp.mean(hc * hc))
        h = h / (std + 1e-5)
        h = h * lng_r[...] + lnb_r[...]
        h = h + _mmT(xres_r[...], wres_r[...]) + bres_r[...]
        h = jnp.maximum(h, 0.2 * h)
        t = jnp.tanh(_mmT(h, wg1_r[...]) + bg1_r[...])
        gate = jnp.sum(t * wg2_r[...], axis=1, keepdims=True) + bg2_r[0]
        ge = jnp.exp(gate)
        onehot = (batch_r[...]
                  == lax.broadcasted_iota(_i32, (G, 1), 0)).astype(_f32)
        gden = lax.dot_general(onehot, ge, (((1,), (0,)), ((), ())),
                               preferred_element_type=_f32,
                               precision=lax.Precision.HIGHEST)
        u = lax.dot_general(onehot, ge * h, (((1,), (0,)), ((), ())),
                            preferred_element_type=_f32,
                            precision=lax.Precision.HIGHEST)
        emb = u / (gden + 1e-16)
        outs = []
        for o in range(O):
            z = _mmT(emb, wh1_r[o]) + bh1_r[o]
            z = z * jax.nn.sigmoid(z)
            z = z * gh_r[o] + bh_r[o]
            outs.append(jnp.sum(z * wh2_r[o], axis=1, keepdims=True)
                        + bh2_r[o, 0])
        out_r[...] = jnp.concatenate(outs, axis=1)

    return pl.pallas_call(
        body,
        out_shape=jax.ShapeDtypeStruct((G, O), _f32),
    )(h_gat, gin_parts, x_res, batch2d, W1, b1, g1, bb1, lng, lnb,
      Wres, bres, Wg1, bg1, Wg2, bg2, Wh1, bh1, gh, bh, Wh2, bh2)


def kernel(x, edge_index, batch, Wp, bp, g0, b0, Wl, bl, Wr, br, att, bgat,
           W1, b1, g1, bb1, lng, lnb, Wres, bres, Wg1, bg1, Wg2, bg2,
           Wh1, bh1, gh, bh, Wh2, bh2):
    src = edge_index[0]
    dst = edge_index[1]
    h0, xl, xr = _tc_proj(x, Wp, bp, g0, b0, Wl, bl, Wr, br)
    ex, den_parts = _sc_scores(xl, xr, src, dst, att.reshape(H * DH))
    rden = _tc_rden(den_parts)
    alpha = _sc_alpha(ex, rden, dst)
    gat_parts = _sc_gat_agg(xl, src, dst, alpha)
    h_gat = _tc_combine(gat_parts, bgat)
    gin_parts = _sc_gin_agg(h_gat, src, dst)
    out = _tc_tail(h_gat, gin_parts, h0, batch.reshape(1, N), W1, b1, g1,
                   bb1, lng, lnb, Wres, bres, Wg1, bg1, Wg2, bg2,
                   Wh1, bh1, gh, bh, Wh2, bh2)
    return out.reshape(G, O, 1)
